# Initial kernel scaffold; baseline (speedup 1.0000x reference)
import functools

import jax
import jax.numpy as jnp
from jax import lax
from jax.experimental import pallas as pl
from jax.experimental.pallas import tpu as pltpu
from jax.experimental.pallas import tpu_sc as plsc

NC = 2
NS = 16
NW = NC * NS
CHUNK = 128


def _gather_call(n_total, dim):
    assert n_total % (NW * CHUNK) == 0
    n_per_w = n_total // NW
    n_chunks = n_per_w // CHUNK

    mesh = plsc.VectorSubcoreMesh(core_axis_name="c", subcore_axis_name="s")

    @functools.partial(
        pl.kernel,
        mesh=mesh,
        out_type=jax.ShapeDtypeStruct((n_total, dim), jnp.float32),
        scratch_types=[
            pltpu.VMEM((n_chunks, CHUNK), jnp.int32),
            pltpu.VMEM((CHUNK, dim), jnp.float32),
            pltpu.SemaphoreType.DMA,
        ],
    )
    def k(table_hbm, idx_hbm, out_hbm, idx_v, rows_v, sem):
        wid = lax.axis_index("s") * NC + lax.axis_index("c")
        row0 = wid * n_chunks
        pltpu.sync_copy(idx_hbm.at[pl.ds(row0, n_chunks)], idx_v)

        def body(g, _):
            pltpu.async_copy(table_hbm.at[idx_v.at[g]], rows_v, sem).wait()
            base = (row0 + g) * CHUNK
            pltpu.sync_copy(rows_v, out_hbm.at[pl.ds(base, CHUNK)])
            return 0

        lax.fori_loop(0, n_chunks, body, 0)

    return k


def kernel(embedding, x):
    dim = embedding.shape[-1]
    x_flat = x.reshape(-1).astype(jnp.int32)
    n_total = x_flat.shape[0]
    idx2d = x_flat.reshape(n_total // CHUNK, CHUNK)
    out = _gather_call(n_total, dim)(embedding, idx2d)
    return out.reshape(x.shape + (dim,))


# speedup vs baseline: 3.5372x; 3.5372x over previous
import functools

import jax
import jax.numpy as jnp
from jax import lax
from jax.experimental import pallas as pl
from jax.experimental.pallas import tpu as pltpu
from jax.experimental.pallas import tpu_sc as plsc

NC = 2
NS = 16
NW = NC * NS
CHUNK = 128


def _gather_call(n_total, dim):
    assert n_total % (NW * CHUNK) == 0
    n_per_w = n_total // NW
    n_chunks = n_per_w // CHUNK

    mesh = plsc.VectorSubcoreMesh(core_axis_name="c", subcore_axis_name="s")

    @functools.partial(
        pl.kernel,
        mesh=mesh,
        out_type=jax.ShapeDtypeStruct((n_total, dim), jnp.float32),
        scratch_types=[
            pltpu.VMEM((n_chunks, CHUNK), jnp.int32),
            pltpu.VMEM((CHUNK, dim), jnp.float32),
            pltpu.SemaphoreType.DMA,
        ],
        compiler_params=pltpu.CompilerParams(use_tc_tiling_on_sc=False),
    )
    def k(table_hbm, idx_hbm, out_hbm, idx_v, rows_v, sem):
        wid = lax.axis_index("s") * NC + lax.axis_index("c")
        row0 = wid * n_chunks
        pltpu.sync_copy(idx_hbm.at[pl.ds(row0, n_chunks)], idx_v)

        def body(g, _):
            pltpu.async_copy(table_hbm.at[idx_v.at[g]], rows_v, sem).wait()
            base = (row0 + g) * CHUNK
            pltpu.sync_copy(rows_v, out_hbm.at[pl.ds(base, CHUNK)])
            return 0

        lax.fori_loop(0, n_chunks, body, 0)

    return k


def kernel(embedding, x):
    dim = embedding.shape[-1]
    x_flat = x.reshape(-1).astype(jnp.int32)
    n_total = x_flat.shape[0]
    idx2d = x_flat.reshape(n_total // CHUNK, CHUNK)
    out = _gather_call(n_total, dim)(embedding, idx2d)
    return out.reshape(x.shape + (dim,))


# speedup vs baseline: 4.2618x; 1.2048x over previous
import functools

import jax
import jax.numpy as jnp
from jax import lax
from jax.experimental import pallas as pl
from jax.experimental.pallas import tpu as pltpu
from jax.experimental.pallas import tpu_sc as plsc

NC = 2
NS = 16
NW = NC * NS
CHUNK = 128


RING = 4
PREF = 2


def _gather_call(n_total, dim):
    assert n_total % (NW * CHUNK) == 0
    n_per_w = n_total // NW
    n_chunks = n_per_w // CHUNK
    assert n_chunks % RING == 0 and PREF <= RING

    mesh = plsc.VectorSubcoreMesh(core_axis_name="c", subcore_axis_name="s")

    @functools.partial(
        pl.kernel,
        mesh=mesh,
        out_type=jax.ShapeDtypeStruct((n_total, dim), jnp.float32),
        scratch_types=[
            pltpu.VMEM((n_chunks, CHUNK), jnp.int32),
            pltpu.VMEM((RING, CHUNK, dim), jnp.float32),
            pltpu.SemaphoreType.DMA((RING,)),
            pltpu.SemaphoreType.DMA((RING,)),
        ],
        compiler_params=pltpu.CompilerParams(use_tc_tiling_on_sc=False),
    )
    def k(table_hbm, idx_hbm, out_hbm, idx_v, rows_v, gsem, osem):
        wid = lax.axis_index("s") * NC + lax.axis_index("c")
        row0 = wid * n_chunks
        pltpu.sync_copy(idx_hbm.at[pl.ds(row0, n_chunks)], idx_v)

        def gather(g, b):
            return pltpu.make_async_copy(
                table_hbm.at[idx_v.at[g]], rows_v.at[b], gsem.at[b])

        def write(g, b):
            return pltpu.make_async_copy(
                rows_v.at[b], out_hbm.at[pl.ds((row0 + g) * CHUNK, CHUNK)],
                osem.at[b])

        for g in range(PREF):
            gather(g, g % RING).start()

        def outer(t, _):
            g0 = t * RING
            for j in range(RING):
                g = g0 + j
                bp = (j + PREF) % RING
                gp = g + PREF

                @pl.when(gp < n_chunks)
                def _issue():
                    @pl.when(gp >= RING)
                    def _drain():
                        write(g, bp).wait()
                    gather(gp, bp).start()

                gather(g, j).wait()
                write(g, j).start()
            return 0

        lax.fori_loop(0, n_chunks // RING, outer, 0)
        for j in range(RING):
            write(0, j).wait()

    return k


def kernel(embedding, x):
    dim = embedding.shape[-1]
    x_flat = x.reshape(-1).astype(jnp.int32)
    n_total = x_flat.shape[0]
    idx2d = x_flat.reshape(n_total // CHUNK, CHUNK)
    out = _gather_call(n_total, dim)(embedding, idx2d)
    return out.reshape(x.shape + (dim,))
